# baseline (speedup 1.0000x reference)
import functools

import jax
import jax.numpy as jnp
from jax import lax
from jax.experimental import pallas as pl
from jax.experimental.pallas import tpu as pltpu
from jax.experimental.pallas import tpu_sc as plsc

_N = 16384
_V = 1000000
_EMB = 10
_OUT = 4
_NC = 2
_NS = 16
_NW = _NC * _NS
_ROWS_PER_TILE = _N // _NS
_CHUNK_WORDS = (_N * _OUT) // _NW
_GCH = 128
_NGCH = _ROWS_PER_TILE // _GCH
_WB = _OUT * _EMB + 16

_mesh = plsc.VectorSubcoreMesh(core_axis_name="c", subcore_axis_name="s")


@functools.partial(
    pl.kernel,
    out_type=jax.ShapeDtypeStruct((_N * _OUT,), jnp.float32),
    mesh=_mesh,
    compiler_params=pltpu.CompilerParams(
        needs_layout_passes=False, use_tc_tiling_on_sc=False),
    scratch_types=[
        pltpu.VMEM((_ROWS_PER_TILE,), jnp.int32),
        pltpu.VMEM((_EMB, _ROWS_PER_TILE), jnp.int32),
        pltpu.VMEM((_EMB, _ROWS_PER_TILE), jnp.float32),
        pltpu.VMEM((16, 16), jnp.float32),
        pltpu.VMEM((1, 16), jnp.int32),
        pltpu.VMEM((_WB,), jnp.float32),
        pltpu.VMEM((16,), jnp.float32),
        pltpu.VMEM((16 * 16,), jnp.float32),
        pltpu.VMEM((_CHUNK_WORDS,), jnp.float32),
        pltpu.VMEM_SHARED((16, 16), jnp.float32),
        pltpu.SemaphoreType.DMA,
    ],
)
def _sc_bag(text_hbm, wb_hbm, tblf_hbm, out_hbm,
            idx_v, idxc_v, vals_v, accbuf_v, cid2_v, wb_v, z16_v,
            accflat_v, chunk_v, acc_sh, gsem):
    c = lax.axis_index("c")
    s = lax.axis_index("s")
    wid = s * _NC + c
    chunk = (_NW - 1) - wid
    owner = jnp.logical_and(c == 0, s == 0)
    iota = lax.broadcasted_iota(jnp.int32, (16,), 0)

    pltpu.sync_copy(wb_hbm, wb_v)

    @pl.when(owner)
    def _():
        z16_v[...] = jnp.zeros((16,), jnp.float32)
        for r in range(16):
            pltpu.sync_copy(z16_v, acc_sh.at[r])

    plsc.subcore_barrier()

    @pl.when(c == 0)
    def _():
        pltpu.sync_copy(
            text_hbm.at[pl.ds(s * _ROWS_PER_TILE, _ROWS_PER_TILE)], idx_v)
        for blk in range(_ROWS_PER_TILE // 16):
            t = idx_v[pl.ds(blk * 16, 16)]
            for e in range(_EMB):
                idxc_v[e, pl.ds(blk * 16, 16)] = t + e * _V
        copies = []
        for e in range(_EMB):
            for g in range(_NGCH):
                copies.append(pltpu.async_copy(
                    tblf_hbm.at[idxc_v.at[e, pl.ds(g * _GCH, _GCH)]],
                    vals_v.at[e, pl.ds(g * _GCH, _GCH)], gsem))
        for cp in copies:
            cp.wait()
        zv = jnp.zeros((16,), jnp.float32)
        for e in range(_EMB):
            acc = zv
            for blk in range(_ROWS_PER_TILE // 16):
                acc = acc + vals_v[e, pl.ds(blk * 16, 16)]
            accbuf_v[e, pl.ds(0, 16)] = acc
        for e in range(_EMB, 16):
            accbuf_v[e, pl.ds(0, 16)] = zv
        cid2_v[0, pl.ds(0, 16)] = iota
        pltpu.sync_copy(accbuf_v, acc_sh.at[cid2_v.at[0]], add=True)

    bpat = wb_v[pl.ds(_OUT * _EMB, 16)]
    for i in range(_CHUNK_WORDS // 16):
        chunk_v[pl.ds(i * 16, 16)] = bpat

    @pl.when(jnp.logical_not(owner))
    def _():
        pltpu.sync_copy(chunk_v,
                        out_hbm.at[pl.ds(chunk * _CHUNK_WORDS, _CHUNK_WORDS)])

    plsc.subcore_barrier()

    @pl.when(owner)
    def _():
        for r in range(16):
            pltpu.sync_copy(acc_sh.at[r], accflat_v.at[pl.ds(16 * r, 16)])
        cs = jnp.zeros((16,), jnp.float32)
        for e in range(_EMB):
            ce = jnp.sum(accflat_v[pl.ds(16 * e, 16)])
            cs = jnp.where(iota == e, ce, cs)
        mean = cs * (1.0 / _N)
        yv = bpat
        for o in range(_OUT):
            wv = wb_v[pl.ds(o * _EMB, 16)]
            y = jnp.sum(mean * wv) + bpat[o]
            yv = jnp.where(iota == 16 - _OUT + o, y, yv)
        chunk_v[pl.ds(_CHUNK_WORDS - 16, 16)] = yv
        pltpu.sync_copy(
            chunk_v,
            out_hbm.at[pl.ds((_NW - 1) * _CHUNK_WORDS, _CHUNK_WORDS)])


@jax.jit
def kernel(text, table, W, b):
    wb = jnp.concatenate([
        W.reshape(-1).astype(jnp.float32),
        jnp.tile(b.reshape(-1).astype(jnp.float32), 4),
    ])
    tblf = table.astype(jnp.float32).T.reshape(-1)
    flat = _sc_bag(text.astype(jnp.int32), wb, tblf)
    return flat.reshape(_N, _OUT)


# speedup vs baseline: 9.2988x; 9.2988x over previous
import functools

import jax
import jax.numpy as jnp
from jax import lax
from jax.experimental import pallas as pl
from jax.experimental.pallas import tpu as pltpu
from jax.experimental.pallas import tpu_sc as plsc

_N = 16384
_V = 1000000
_E = 10
_OUT = 4
_BK = 76928
_NBK = 13
_HP = _BK * _NBK
_SLICE = _HP // 16
_ZCH = 4096

_mesh = plsc.VectorSubcoreMesh(core_axis_name="c", subcore_axis_name="s")


@functools.partial(
    pl.kernel,
    out_type=jax.ShapeDtypeStruct((_HP,), jnp.float32),
    mesh=_mesh,
    compiler_params=pltpu.CompilerParams(
        needs_layout_passes=False, use_tc_tiling_on_sc=False),
    scratch_types=[
        pltpu.VMEM((_ZCH,), jnp.float32),
        pltpu.VMEM((128,), jnp.float32),
        pltpu.VMEM((8, 128), jnp.int32),
        pltpu.VMEM_SHARED((_HP,), jnp.float32),
    ],
)
def _sc_hist(text_hbm, hist_hbm, zbuf_v, ones_v, idx2d_v, hist_sp):
    c = lax.axis_index("c")
    s = lax.axis_index("s")

    @pl.when(c == 0)
    def _():
        zv = jnp.zeros((16,), jnp.float32)
        for i in range(_ZCH // 16):
            zbuf_v[pl.ds(i * 16, 16)] = zv
        for i in range(8):
            ones_v[pl.ds(i * 16, 16)] = jnp.ones((16,), jnp.float32)
        base = s * _SLICE
        for i in range(15):
            pltpu.sync_copy(zbuf_v, hist_sp.at[pl.ds(base + i * _ZCH, _ZCH)])
        pltpu.sync_copy(zbuf_v.at[pl.ds(0, _SLICE - 15 * _ZCH)],
                        hist_sp.at[pl.ds(base + 15 * _ZCH,
                                         _SLICE - 15 * _ZCH)])
        for g in range(8):
            pltpu.sync_copy(text_hbm.at[pl.ds(s * 1024 + g * 128, 128)],
                            idx2d_v.at[g])

    plsc.subcore_barrier()

    @pl.when(c == 0)
    def _():
        for g in range(8):
            pltpu.sync_copy(ones_v, hist_sp.at[idx2d_v.at[g]], add=True)

    plsc.subcore_barrier()

    @pl.when(c == 0)
    def _():
        base = s * _SLICE
        pltpu.sync_copy(hist_sp.at[pl.ds(base, _SLICE)],
                        hist_hbm.at[pl.ds(base, _SLICE)])


def _tc_body(h_ref, tb_ref, wpack_ref, bpack_ref, out_ref, acc_ref):
    k = pl.program_id(0)

    @pl.when(k == 0)
    def _():
        acc_ref[...] = jnp.zeros((1, _E), jnp.float32)

    h = h_ref[...]
    dn = (((1,), (1,)), ((), ()))

    @pl.when(k < _NBK - 1)
    def _():
        acc_ref[...] += jax.lax.dot_general(
            h, tb_ref[...], dn, preferred_element_type=jnp.float32)

    @pl.when(k == _NBK - 1)
    def _():
        valid = _V - (_NBK - 1) * _BK
        m = lax.broadcasted_iota(jnp.int32, (_E, _BK), 1) < valid
        tb = jnp.where(m, tb_ref[...], 0.0)
        acc = acc_ref[...] + jax.lax.dot_general(
            h, tb, dn, preferred_element_type=jnp.float32)
        mean = acc * (1.0 / _N)
        yrow = bpack_ref[...] + jax.lax.dot_general(
            mean, wpack_ref[...], (((1,), (0,)), ((), ())),
            preferred_element_type=jnp.float32)
        rowio = lax.broadcasted_iota(jnp.int32, (512, 128), 0)
        base = jnp.broadcast_to(bpack_ref[...], (512, 128))
        yb = jnp.broadcast_to(yrow, (512, 128))
        out_ref[...] = jnp.where(rowio == 511, yb, base)


_tc_mv = pl.pallas_call(
    _tc_body,
    grid=(_NBK,),
    in_specs=[
        pl.BlockSpec((1, _BK), lambda k: (0, k)),
        pl.BlockSpec((_E, _BK), lambda k: (0, k)),
        pl.BlockSpec((_E, 128), lambda k: (0, 0)),
        pl.BlockSpec((1, 128), lambda k: (0, 0)),
    ],
    out_specs=pl.BlockSpec((512, 128), lambda k: (0, 0)),
    out_shape=jax.ShapeDtypeStruct((512, 128), jnp.float32),
    scratch_shapes=[pltpu.VMEM((1, _E), jnp.float32)],
)


@jax.jit
def kernel(text, table, W, b):
    Wf = W.astype(jnp.float32)
    bf = b.astype(jnp.float32)
    hist = _sc_hist(text.astype(jnp.int32))
    wpack = jnp.zeros((_E, 128), jnp.float32)
    for o in range(_OUT):
        wpack = wpack.at[:, 128 - _OUT + o].set(Wf[o, :])
    bpack = jnp.tile(bf, 128 // _OUT).reshape(1, 128)
    out2d = _tc_mv(hist.reshape(1, _HP), table.astype(jnp.float32).T,
                   wpack, bpack)
    return out2d.reshape(_N, _OUT)


# speedup vs baseline: 14.8977x; 1.6021x over previous
import functools

import jax
import jax.numpy as jnp
from jax import lax
from jax.experimental import pallas as pl
from jax.experimental.pallas import tpu as pltpu
from jax.experimental.pallas import tpu_sc as plsc

_N = 16384
_V = 1000000
_E = 10
_OUT = 4
_BK = 125056
_NBK = 8
_HP = _BK * _NBK
_SLICE = _HP // 16
_ZCH = 4096

_mesh = plsc.VectorSubcoreMesh(core_axis_name="c", subcore_axis_name="s")


@functools.partial(
    pl.kernel,
    out_type=jax.ShapeDtypeStruct((_HP,), jnp.float32),
    mesh=_mesh,
    compiler_params=pltpu.CompilerParams(
        needs_layout_passes=False, use_tc_tiling_on_sc=False),
    scratch_types=[
        pltpu.VMEM((_ZCH,), jnp.float32),
        pltpu.VMEM((128,), jnp.float32),
        pltpu.VMEM((8, 128), jnp.int32),
        pltpu.VMEM_SHARED((_HP,), jnp.float32),
    ],
)
def _sc_hist(text_hbm, hist_hbm, zbuf_v, ones_v, idx2d_v, hist_sp):
    c = lax.axis_index("c")
    s = lax.axis_index("s")

    @pl.when(c == 0)
    def _():
        zv = jnp.zeros((16,), jnp.float32)
        for i in range(_ZCH // 16):
            zbuf_v[pl.ds(i * 16, 16)] = zv
        for i in range(8):
            ones_v[pl.ds(i * 16, 16)] = jnp.ones((16,), jnp.float32)
        base = s * _SLICE
        nfull = _SLICE // _ZCH
        for i in range(nfull):
            pltpu.sync_copy(zbuf_v, hist_sp.at[pl.ds(base + i * _ZCH, _ZCH)])
        rem = _SLICE - nfull * _ZCH
        if rem:
            pltpu.sync_copy(zbuf_v.at[pl.ds(0, rem)],
                            hist_sp.at[pl.ds(base + nfull * _ZCH, rem)])
        for g in range(8):
            pltpu.sync_copy(text_hbm.at[pl.ds(s * 1024 + g * 128, 128)],
                            idx2d_v.at[g])

    plsc.subcore_barrier()

    @pl.when(c == 0)
    def _():
        for g in range(8):
            pltpu.sync_copy(ones_v, hist_sp.at[idx2d_v.at[g]], add=True)

    plsc.subcore_barrier()

    @pl.when(c == 0)
    def _():
        base = s * _SLICE
        pltpu.sync_copy(hist_sp.at[pl.ds(base, _SLICE)],
                        hist_hbm.at[pl.ds(base, _SLICE)])


def _tc_body(h_ref, tb_ref, w_ref, b4_ref, out_ref, acc_ref):
    k = pl.program_id(0)

    @pl.when(k == 0)
    def _():
        acc_ref[...] = jnp.zeros((_E, 1), jnp.float32)

    h = h_ref[...]
    dn = (((1,), (1,)), ((), ()))

    @pl.when(k < _NBK - 1)
    def _():
        acc_ref[...] += jax.lax.dot_general(
            tb_ref[...], h, dn, preferred_element_type=jnp.float32)

    @pl.when(k == _NBK - 1)
    def _():
        valid = _V - (_NBK - 1) * _BK
        m = lax.broadcasted_iota(jnp.int32, (_E, _BK), 1) < valid
        tb = jnp.where(m, tb_ref[...], 0.0)
        acc = acc_ref[...] + jax.lax.dot_general(
            tb, h, dn, preferred_element_type=jnp.float32)
        y4 = jax.lax.dot_general(
            w_ref[...], acc, (((1,), (0,)), ((), ())),
            preferred_element_type=jnp.float32) * (1.0 / _N)
        bcol = b4_ref[:, 0:1]
        colio = lax.broadcasted_iota(jnp.int32, (_OUT, _N), 1)
        out_ref[...] = jnp.where(
            colio == _N - 1,
            jnp.broadcast_to(y4 + bcol, (_OUT, _N)),
            jnp.broadcast_to(bcol, (_OUT, _N)))


_tc_mv = pl.pallas_call(
    _tc_body,
    grid=(_NBK,),
    in_specs=[
        pl.BlockSpec((1, _BK), lambda k: (0, k)),
        pl.BlockSpec((_E, _BK), lambda k: (0, k)),
        pl.BlockSpec((_OUT, _E), lambda k: (0, 0)),
        pl.BlockSpec((_OUT, 128), lambda k: (0, 0)),
    ],
    out_specs=pl.BlockSpec((_OUT, _N), lambda k: (0, 0)),
    out_shape=jax.ShapeDtypeStruct((_OUT, _N), jnp.float32),
    scratch_shapes=[pltpu.VMEM((_E, 1), jnp.float32)],
)


@jax.jit
def kernel(text, table, W, b):
    hist = _sc_hist(text.astype(jnp.int32))
    b4 = jnp.broadcast_to(b.astype(jnp.float32)[:, None], (_OUT, 128))
    out_t = _tc_mv(hist.reshape(1, _HP), table.astype(jnp.float32).T,
                   W.astype(jnp.float32), b4)
    return out_t.T
